# baseline (speedup 1.0000x reference)
import functools

import jax
import jax.numpy as jnp
from jax import lax
from jax.experimental import pallas as pl
from jax.experimental.pallas import tpu as pltpu
from jax.experimental.pallas import tpu_sc as plsc

N_NODES = 10000
N_EDGES = 320000
BATCH_SIZE = 64
NODE_DIM = 128
HIDDEN = 256
EMBED = 128
GLOBAL = 2048
NUM_CLASSES = 16

NC, NS = 2, 16
NW = NC * NS
EDGES_PER_W = N_EDGES // NW
CHUNK = 80
N_CHUNKS = EDGES_PER_W // CHUNK
STRIPE = 624
TAIL = N_NODES - NS * STRIPE
WB = 48
PAD_DIM = 144

_MESH = plsc.VectorSubcoreMesh(core_axis_name="c", subcore_axis_name="s")


def _zero_vmem(ref, rows, cols):
    zero = jnp.zeros((16,), jnp.float32)

    def row(i, carry):
        for j in range(cols // 16):
            ref[i, pl.ds(j * 16, 16)] = zero
        return carry

    lax.fori_loop(0, rows, row, 0)


def _striped(s, copy_fn):
    for k in range(STRIPE // WB):
        copy_fn(s * STRIPE + k * WB, WB)

    @pl.when(s == 0)
    def _():
        copy_fn(NS * STRIPE, TAIL)


def _zero_then(s, wb_v, acc_sp):
    _zero_vmem(wb_v, WB, NODE_DIM)
    _striped(s, lambda o, n: pltpu.sync_copy(wb_v.at[pl.ds(0, n)],
                                             acc_sp.at[pl.ds(o, n)]))


def _writeback(s, c, wb_v, acc_sp, out_hbm):
    def wb(o, n):
        pltpu.sync_copy(acc_sp.at[pl.ds(o, n)], wb_v.at[pl.ds(0, n)])
        pltpu.sync_copy(wb_v.at[pl.ds(0, n)],
                        out_hbm.at[pl.ds(c * N_NODES + o, n)])

    _striped(s, wb)


def _sc_agg_body(with_deg, refs):
    if with_deg:
        (feat_hbm, src_hbm, dst_hbm, agg_out, deg_out,
         src_a, src_b, dst_i, rows_a, rows_b, wb_v,
         gsem, isem, ssem, acc_sp) = refs
    else:
        (feat_hbm, src_hbm, dst_hbm, agg_out,
         src_a, src_b, dst_i, rows_a, rows_b, wb_v,
         gsem, isem, ssem, acc_sp) = refs
    c = lax.axis_index("c")
    s = lax.axis_index("s")
    wid = c * NS + s
    ebase = wid * EDGES_PER_W

    def fill(i, buf):
        pltpu.async_copy(src_hbm.at[pl.ds(ebase + i * CHUNK, CHUNK)],
                         buf, isem)

    def drain_fill(i, buf):
        pltpu.make_async_copy(src_hbm.at[pl.ds(ebase + i * CHUNK, CHUNK)],
                              buf, isem).wait()

    def gather(buf, rows):
        pltpu.async_copy(feat_hbm.at[buf], rows, gsem)

    def drain_gather(buf, rows):
        pltpu.make_async_copy(feat_hbm.at[buf], rows, gsem).wait()

    def scatter(i, rows):
        pltpu.async_copy(rows, acc_sp.at[dst_i.at[i]], ssem, add=True)

    def drain_scatter(i, rows):
        pltpu.make_async_copy(rows, acc_sp.at[dst_i.at[i]], ssem).wait()

    fill(0, src_a)
    fill(1, src_b)
    pltpu.sync_copy(dst_hbm.at[wid], dst_i)
    drain_fill(0, src_a)
    gather(src_a, rows_a)
    _zero_then(s, wb_v, acc_sp)
    plsc.subcore_barrier()

    def step(i, sa, sb, ra, rb, first=False):
        drain_gather(sa, ra)
        fill(i + 2, sa)
        drain_fill(i + 1, sb)
        if not first:
            drain_scatter(i - 1, rb)
        gather(sb, rb)
        scatter(i, ra)

    step(0, src_a, src_b, rows_a, rows_b, first=True)

    def pair(k, carry):
        i = 2 * k + 1
        step(i, src_b, src_a, rows_b, rows_a)
        step(i + 1, src_a, src_b, rows_a, rows_b)
        return carry

    lax.fori_loop(0, (N_CHUNKS - 1) // 2, pair, 0)
    drain_scatter(N_CHUNKS - 1, rows_a)
    drain_gather(src_b, rows_b)
    drain_fill(N_CHUNKS + 1, src_a)
    plsc.subcore_barrier()
    _writeback(s, c, wb_v, acc_sp, agg_out)

    if with_deg:
        plsc.subcore_barrier()
        _zero_then(s, wb_v, acc_sp)
        _zero_vmem(rows_a, CHUNK, NODE_DIM)
        one16 = jnp.where(lax.iota(jnp.int32, 16) == 0, 1.0, 0.0)

        def onerow(i, carry):
            rows_a[i, pl.ds(0, 16)] = one16
            return carry

        lax.fori_loop(0, CHUNK, onerow, 0)
        plsc.subcore_barrier()

        def dscatter(i):
            return pltpu.async_copy(rows_a, acc_sp.at[dst_i.at[i]], ssem,
                                    add=True)

        def drain_dscatter(i):
            pltpu.make_async_copy(rows_a, acc_sp.at[dst_i.at[i]], ssem).wait()

        dscatter(0)

        def dchunk(i, carry):
            dscatter(i + 1)
            drain_dscatter(i)
            return carry

        lax.fori_loop(0, N_CHUNKS - 1, dchunk, 0)
        drain_dscatter(N_CHUNKS - 1)
        plsc.subcore_barrier()
        _writeback(s, c, wb_v, acc_sp, deg_out)


_AGG_SCRATCH = [
    pltpu.VMEM((CHUNK,), jnp.int32),
    pltpu.VMEM((CHUNK,), jnp.int32),
    pltpu.VMEM((N_CHUNKS, CHUNK), jnp.int32),
    pltpu.VMEM((CHUNK, NODE_DIM), jnp.float32),
    pltpu.VMEM((CHUNK, NODE_DIM), jnp.float32),
    pltpu.VMEM((WB, NODE_DIM), jnp.float32),
    pltpu.SemaphoreType.DMA,
    pltpu.SemaphoreType.DMA,
    pltpu.SemaphoreType.DMA,
    pltpu.VMEM_SHARED((N_NODES, NODE_DIM), jnp.float32),
]


def _sc_agg(feat, src, dst):
    return pl.kernel(
        lambda *refs: _sc_agg_body(False, refs),
        out_type=jax.ShapeDtypeStruct((NC * N_NODES, NODE_DIM), jnp.float32),
        mesh=_MESH,
        scratch_types=_AGG_SCRATCH,
        name="sc_agg",
    )(feat, src, dst)


def _sc_agg_deg(feat, src, dst):
    return pl.kernel(
        lambda *refs: _sc_agg_body(True, refs),
        out_type=[
            jax.ShapeDtypeStruct((NC * N_NODES, NODE_DIM), jnp.float32),
            jax.ShapeDtypeStruct((NC * N_NODES, NODE_DIM), jnp.float32),
        ],
        mesh=_MESH,
        scratch_types=_AGG_SCRATCH,
        name="sc_agg_deg",
    )(feat, src, dst)


_DOT = functools.partial(jnp.dot, preferred_element_type=jnp.float32,
                         precision=lax.Precision.HIGHEST)


def _tc1_body(x_ref, aggp_ref, degp_ref, w1l_ref, b1_ref, w1r_ref,
              w2l_ref, w2r_ref, y2_ref, r2_ref, rdeg_ref):
    deg = degp_ref[0, :, 0] + degp_ref[1, :, 0]
    rdeg = 1.0 / jnp.maximum(deg, 1.0)
    agg = aggp_ref[0] + aggp_ref[1]
    mean = agg * rdeg[:, None]
    h1 = _DOT(mean, w1l_ref[...]) + b1_ref[...] + _DOT(x_ref[...], w1r_ref[...])
    h1 = jnp.maximum(h1, 0.0)
    y2_ref[...] = _DOT(h1, w2l_ref[...])
    r2_ref[...] = _DOT(h1, w2r_ref[...])
    rdeg_ref[...] = rdeg[:, None]


def _tc1(x, agg1p, degp, w1l, b1, w1r, w2l, w2r):
    blk = 1000
    grid = (N_NODES // blk,)
    return pl.pallas_call(
        _tc1_body,
        grid=grid,
        in_specs=[
            pl.BlockSpec((blk, NODE_DIM), lambda i: (i, 0)),
            pl.BlockSpec((NC, blk, NODE_DIM), lambda i: (0, i, 0)),
            pl.BlockSpec((NC, blk, NODE_DIM), lambda i: (0, i, 0)),
            pl.BlockSpec((NODE_DIM, HIDDEN), lambda i: (0, 0)),
            pl.BlockSpec((HIDDEN,), lambda i: (0,)),
            pl.BlockSpec((NODE_DIM, HIDDEN), lambda i: (0, 0)),
            pl.BlockSpec((HIDDEN, EMBED), lambda i: (0, 0)),
            pl.BlockSpec((HIDDEN, EMBED), lambda i: (0, 0)),
        ],
        out_specs=[
            pl.BlockSpec((blk, EMBED), lambda i: (i, 0)),
            pl.BlockSpec((blk, EMBED), lambda i: (i, 0)),
            pl.BlockSpec((blk, 1), lambda i: (i, 0)),
        ],
        out_shape=[
            jax.ShapeDtypeStruct((N_NODES, EMBED), jnp.float32),
            jax.ShapeDtypeStruct((N_NODES, EMBED), jnp.float32),
            jax.ShapeDtypeStruct((N_NODES, 1), jnp.float32),
        ],
        name="tc1_dense",
    )(x, agg1p, degp, w1l, b1, w1r, w2l, w2r)


def _tc2_body(agg2p_ref, rdeg_ref, r2_ref, b2_ref, batch_ref, gf_ref,
              clsw_ref, clsb_ref, out_ref):
    rdeg = rdeg_ref[...]
    h2 = (agg2p_ref[0] + agg2p_ref[1]) * rdeg + b2_ref[...] + r2_ref[...]
    h2 = jnp.maximum(h2, 0.0)
    seg = lax.broadcasted_iota(jnp.int32, (BATCH_SIZE, N_NODES), 0)
    mask = (seg == batch_ref[...]).astype(jnp.float32)
    pooled = _DOT(mask, h2)
    cnt = jnp.sum(mask, axis=1)
    gnn = pooled / jnp.maximum(cnt, 1.0)[:, None]
    logits = (_DOT(gnn, clsw_ref[:EMBED, :])
              + _DOT(gf_ref[...], clsw_ref[EMBED:, :])
              + clsb_ref[...])
    out_ref[...] = logits


def _tc2(agg2p, rdeg, r2, b2, batch2d, gf, clsw, clsb):
    return pl.pallas_call(
        _tc2_body,
        out_shape=jax.ShapeDtypeStruct((BATCH_SIZE, NUM_CLASSES), jnp.float32),
        name="tc2_pool_cls",
    )(agg2p, rdeg, r2, b2, batch2d, gf, clsw, clsb)


def kernel(x, edge_index, batch, global_feat, conv1_Wl, conv1_bl, conv1_Wr,
           conv2_Wl, conv2_bl, conv2_Wr, cls_W, cls_b):
    src = jnp.concatenate([edge_index[0].astype(jnp.int32),
                           jnp.zeros((2 * CHUNK,), jnp.int32)])
    dst = edge_index[1].astype(jnp.int32).reshape(NW, N_CHUNKS, CHUNK)
    batch2d = batch.astype(jnp.int32).reshape(1, N_NODES)

    agg1p, degp = _sc_agg_deg(x, src, dst)
    agg1p = agg1p.reshape(NC, N_NODES, NODE_DIM)
    degp = degp.reshape(NC, N_NODES, NODE_DIM)
    y2, r2, rdeg = _tc1(x, agg1p, degp, conv1_Wl, conv1_bl, conv1_Wr,
                        conv2_Wl, conv2_Wr)
    agg2p = _sc_agg(y2, src, dst).reshape(NC, N_NODES, EMBED)
    return _tc2(agg2p, rdeg, r2, conv2_bl, batch2d, global_feat, cls_W, cls_b)


# speedup vs baseline: 1.1162x; 1.1162x over previous
import functools

import jax
import jax.numpy as jnp
from jax import lax
from jax.experimental import pallas as pl
from jax.experimental.pallas import tpu as pltpu
from jax.experimental.pallas import tpu_sc as plsc

N_NODES = 10000
N_EDGES = 320000
BATCH_SIZE = 64
NODE_DIM = 128
HIDDEN = 256
EMBED = 128
GLOBAL = 2048
NUM_CLASSES = 16

NC, NS = 2, 16
NW = NC * NS
EDGES_PER_W = N_EDGES // NW
CHUNK = 80
N_CHUNKS = EDGES_PER_W // CHUNK
STRIPE = 624
TAIL = N_NODES - NS * STRIPE
WB = 48
PAD_DIM = 144

_MESH = plsc.VectorSubcoreMesh(core_axis_name="c", subcore_axis_name="s")


def _zero_vmem(ref, rows, cols):
    zero = jnp.zeros((16,), jnp.float32)

    def row(i, carry):
        for j in range(cols // 16):
            ref[i, pl.ds(j * 16, 16)] = zero
        return carry

    lax.fori_loop(0, rows, row, 0)


def _striped(s, copy_fn):
    for k in range(STRIPE // WB):
        copy_fn(s * STRIPE + k * WB, WB)

    @pl.when(s == 0)
    def _():
        copy_fn(NS * STRIPE, TAIL)


def _zero_then(s, wb_v, acc_sp):
    _zero_vmem(wb_v, WB, NODE_DIM)
    _striped(s, lambda o, n: pltpu.sync_copy(wb_v.at[pl.ds(0, n)],
                                             acc_sp.at[pl.ds(o, n)]))


def _writeback(s, c, wb_v, acc_sp, out_hbm):
    def wb(o, n):
        pltpu.sync_copy(acc_sp.at[pl.ds(o, n)], wb_v.at[pl.ds(0, n)])
        pltpu.sync_copy(wb_v.at[pl.ds(0, n)],
                        out_hbm.at[pl.ds(c * N_NODES + o, n)])

    _striped(s, wb)


def _sc_agg_body(with_deg, refs):
    if with_deg:
        (feat_hbm, src_hbm, dst_hbm, agg_out, deg_out,
         src_a, src_b, dst_i, rows_a, rows_b, wb_v,
         gsem, isem, ssem, acc_sp) = refs
    else:
        (feat_hbm, src_hbm, dst_hbm, agg_out,
         src_a, src_b, dst_i, rows_a, rows_b, wb_v,
         gsem, isem, ssem, acc_sp) = refs
    c = lax.axis_index("c")
    s = lax.axis_index("s")
    wid = c * NS + s
    ebase = wid * EDGES_PER_W

    def fill(i, buf):
        pltpu.async_copy(src_hbm.at[pl.ds(ebase + i * CHUNK, CHUNK)],
                         buf, isem)

    def drain_fill(i, buf):
        pltpu.make_async_copy(src_hbm.at[pl.ds(ebase + i * CHUNK, CHUNK)],
                              buf, isem).wait()

    def gather(buf, rows):
        pltpu.async_copy(feat_hbm.at[buf], rows, gsem)

    def drain_gather(buf, rows):
        pltpu.make_async_copy(feat_hbm.at[buf], rows, gsem).wait()

    def scatter(i, rows):
        pltpu.async_copy(rows, acc_sp.at[dst_i.at[i]], ssem, add=True)

    def drain_scatter(i, rows):
        pltpu.make_async_copy(rows, acc_sp.at[dst_i.at[i]], ssem).wait()

    fill(0, src_a)
    fill(1, src_b)
    pltpu.sync_copy(dst_hbm.at[wid], dst_i)
    drain_fill(0, src_a)
    gather(src_a, rows_a)
    _zero_then(s, wb_v, acc_sp)
    plsc.subcore_barrier()

    def step(i, sa, sb, ra, rb, first=False):
        drain_gather(sa, ra)
        fill(i + 2, sa)
        drain_fill(i + 1, sb)
        if not first:
            drain_scatter(i - 1, rb)
        gather(sb, rb)
        scatter(i, ra)

    step(0, src_a, src_b, rows_a, rows_b, first=True)

    def pair(k, carry):
        i = 2 * k + 1
        step(i, src_b, src_a, rows_b, rows_a)
        step(i + 1, src_a, src_b, rows_a, rows_b)
        return carry

    lax.fori_loop(0, (N_CHUNKS - 1) // 2, pair, 0)
    drain_scatter(N_CHUNKS - 1, rows_a)
    drain_gather(src_b, rows_b)
    drain_fill(N_CHUNKS + 1, src_a)
    plsc.subcore_barrier()
    _writeback(s, c, wb_v, acc_sp, agg_out)

    if with_deg:
        plsc.subcore_barrier()
        _zero_then(s, wb_v, acc_sp)
        _zero_vmem(rows_a, CHUNK, NODE_DIM)
        one16 = jnp.where(lax.iota(jnp.int32, 16) == 0, 1.0, 0.0)

        def onerow(i, carry):
            rows_a[i, pl.ds(0, 16)] = one16
            return carry

        lax.fori_loop(0, CHUNK, onerow, 0)
        plsc.subcore_barrier()

        def dscatter(i):
            return pltpu.async_copy(rows_a, acc_sp.at[dst_i.at[i]], ssem,
                                    add=True)

        def drain_dscatter(i):
            pltpu.make_async_copy(rows_a, acc_sp.at[dst_i.at[i]], ssem).wait()

        dscatter(0)

        def dchunk(i, carry):
            dscatter(i + 1)
            drain_dscatter(i)
            return carry

        lax.fori_loop(0, N_CHUNKS - 1, dchunk, 0)
        drain_dscatter(N_CHUNKS - 1)
        plsc.subcore_barrier()
        _writeback(s, c, wb_v, acc_sp, deg_out)


_AGG_SCRATCH = [
    pltpu.VMEM((CHUNK,), jnp.int32),
    pltpu.VMEM((CHUNK,), jnp.int32),
    pltpu.VMEM((N_CHUNKS, CHUNK), jnp.int32),
    pltpu.VMEM((CHUNK, NODE_DIM), jnp.float32),
    pltpu.VMEM((CHUNK, NODE_DIM), jnp.float32),
    pltpu.VMEM((WB, NODE_DIM), jnp.float32),
    pltpu.SemaphoreType.DMA,
    pltpu.SemaphoreType.DMA,
    pltpu.SemaphoreType.DMA,
    pltpu.VMEM_SHARED((N_NODES, NODE_DIM), jnp.float32),
]


def _sc_agg(feat, src, dst):
    return pl.kernel(
        lambda *refs: _sc_agg_body(False, refs),
        out_type=jax.ShapeDtypeStruct((NC * N_NODES, NODE_DIM), jnp.float32),
        mesh=_MESH,
        scratch_types=_AGG_SCRATCH,
        name="sc_agg",
    )(feat, src, dst)


def _sc_agg_deg(feat, src, dst):
    return pl.kernel(
        lambda *refs: _sc_agg_body(True, refs),
        out_type=[
            jax.ShapeDtypeStruct((NC * N_NODES, NODE_DIM), jnp.float32),
            jax.ShapeDtypeStruct((NC * N_NODES, NODE_DIM), jnp.float32),
        ],
        mesh=_MESH,
        scratch_types=_AGG_SCRATCH,
        name="sc_agg_deg",
    )(feat, src, dst)


_DOT = functools.partial(jnp.dot, preferred_element_type=jnp.float32,
                         precision=lax.Precision.DEFAULT)


def _tc1_body(x_ref, aggp_ref, degp_ref, w1l_ref, b1_ref, w1r_ref,
              w2l_ref, w2r_ref, y2_ref, r2_ref, rdeg_ref):
    deg = degp_ref[0, :, 0] + degp_ref[1, :, 0]
    rdeg = 1.0 / jnp.maximum(deg, 1.0)
    agg = aggp_ref[0] + aggp_ref[1]
    mean = agg * rdeg[:, None]
    h1 = _DOT(mean, w1l_ref[...]) + b1_ref[...] + _DOT(x_ref[...], w1r_ref[...])
    h1 = jnp.maximum(h1, 0.0)
    y2_ref[...] = _DOT(h1, w2l_ref[...])
    r2_ref[...] = _DOT(h1, w2r_ref[...])
    rdeg_ref[...] = rdeg[:, None]


def _tc1(x, agg1p, degp, w1l, b1, w1r, w2l, w2r):
    blk = 1000
    grid = (N_NODES // blk,)
    return pl.pallas_call(
        _tc1_body,
        grid=grid,
        in_specs=[
            pl.BlockSpec((blk, NODE_DIM), lambda i: (i, 0)),
            pl.BlockSpec((NC, blk, NODE_DIM), lambda i: (0, i, 0)),
            pl.BlockSpec((NC, blk, NODE_DIM), lambda i: (0, i, 0)),
            pl.BlockSpec((NODE_DIM, HIDDEN), lambda i: (0, 0)),
            pl.BlockSpec((HIDDEN,), lambda i: (0,)),
            pl.BlockSpec((NODE_DIM, HIDDEN), lambda i: (0, 0)),
            pl.BlockSpec((HIDDEN, EMBED), lambda i: (0, 0)),
            pl.BlockSpec((HIDDEN, EMBED), lambda i: (0, 0)),
        ],
        out_specs=[
            pl.BlockSpec((blk, EMBED), lambda i: (i, 0)),
            pl.BlockSpec((blk, EMBED), lambda i: (i, 0)),
            pl.BlockSpec((blk, 1), lambda i: (i, 0)),
        ],
        out_shape=[
            jax.ShapeDtypeStruct((N_NODES, EMBED), jnp.float32),
            jax.ShapeDtypeStruct((N_NODES, EMBED), jnp.float32),
            jax.ShapeDtypeStruct((N_NODES, 1), jnp.float32),
        ],
        name="tc1_dense",
    )(x, agg1p, degp, w1l, b1, w1r, w2l, w2r)


def _tc2_body(agg2p_ref, rdeg_ref, r2_ref, b2_ref, batch_ref, gf_ref,
              clsw_ref, clsb_ref, out_ref):
    rdeg = rdeg_ref[...]
    h2 = (agg2p_ref[0] + agg2p_ref[1]) * rdeg + b2_ref[...] + r2_ref[...]
    h2 = jnp.maximum(h2, 0.0)
    seg = lax.broadcasted_iota(jnp.int32, (BATCH_SIZE, N_NODES), 0)
    mask = (seg == batch_ref[...]).astype(jnp.float32)
    pooled = _DOT(mask, h2)
    cnt = jnp.sum(mask, axis=1)
    gnn = pooled / jnp.maximum(cnt, 1.0)[:, None]
    logits = (_DOT(gnn, clsw_ref[:EMBED, :])
              + _DOT(gf_ref[...], clsw_ref[EMBED:, :])
              + clsb_ref[...])
    out_ref[...] = logits


def _tc2(agg2p, rdeg, r2, b2, batch2d, gf, clsw, clsb):
    return pl.pallas_call(
        _tc2_body,
        out_shape=jax.ShapeDtypeStruct((BATCH_SIZE, NUM_CLASSES), jnp.float32),
        name="tc2_pool_cls",
    )(agg2p, rdeg, r2, b2, batch2d, gf, clsw, clsb)


def kernel(x, edge_index, batch, global_feat, conv1_Wl, conv1_bl, conv1_Wr,
           conv2_Wl, conv2_bl, conv2_Wr, cls_W, cls_b):
    src = jnp.concatenate([edge_index[0].astype(jnp.int32),
                           jnp.zeros((2 * CHUNK,), jnp.int32)])
    dst = edge_index[1].astype(jnp.int32).reshape(NW, N_CHUNKS, CHUNK)
    batch2d = batch.astype(jnp.int32).reshape(1, N_NODES)

    agg1p, degp = _sc_agg_deg(x, src, dst)
    agg1p = agg1p.reshape(NC, N_NODES, NODE_DIM)
    degp = degp.reshape(NC, N_NODES, NODE_DIM)
    y2, r2, rdeg = _tc1(x, agg1p, degp, conv1_Wl, conv1_bl, conv1_Wr,
                        conv2_Wl, conv2_Wr)
    agg2p = _sc_agg(y2, src, dst).reshape(NC, N_NODES, EMBED)
    return _tc2(agg2p, rdeg, r2, conv2_bl, batch2d, global_feat, cls_W, cls_b)
